# Initial kernel scaffold; baseline (speedup 1.0000x reference)
import jax
import jax.numpy as jnp
from jax.experimental import pallas as pl


def kernel(x, adj_matrix, Wl, bl, al, ff_w1, ff_b1, ff_w2, ff_b2, ln1_g, ln1_b, ln2_g, ln2_b, w_out, b_out):
    raise NotImplementedError("write your pallas kernel here")


# speedup vs baseline: 2.2475x; 2.2475x over previous
import functools

import jax
import jax.numpy as jnp
from jax.experimental import pallas as pl
from jax.experimental.pallas import tpu as pltpu

L = 2
H = 8
IN = 256
E = 256
FF = 1024
N = 64
OUT = 128
B = 256

BB = 32
BBO = 128


def _ln(x, g, b):
    m = jnp.mean(x, axis=-1, keepdims=True)
    v = jnp.mean((x - m) ** 2, axis=-1, keepdims=True)
    return (x - m) * jax.lax.rsqrt(v + 1e-5) * g + b


def _mm(a, b):
    return jax.lax.dot_general(a, b, (((1,), (0,)), ((), ())),
                               preferred_element_type=jnp.float32)


def _ghat_body(x_ref, adj_ref, wt_ref, bl_ref, a1t_ref, a2_ref,
               f1t_ref, fb1_ref, f2t_ref, fb2_ref,
               g1_ref, be1_ref, g2_ref, be2_ref, o_ref):
    xb = x_ref[...]
    mask = (adj_ref[...] > 0).astype(jnp.float32)
    for l in range(L):
        xf = xb.reshape(BB * N, IN)
        attn = jnp.zeros((BB, N, E), jnp.float32)
        for hd in range(H):
            h = _mm(xf, wt_ref[l, hd]) + bl_ref[l, hd]
            h3 = h.reshape(BB, N, E)
            hsum = jnp.sum(h3, axis=1)
            s1 = jnp.sum(h3 * a1t_ref[l, hd][None], axis=2)
            p3 = _mm(h, a2_ref[l, hd]).reshape(BB, N, N)
            s2 = jnp.sum(p3 * mask[None], axis=1)
            score = s1 + s2
            hp = score[:, :, None] * hsum[:, None, :]
            attn = attn + jnp.where(hp >= 0, hp, 0.01 * hp)
        xb = _ln(xb + attn, g1_ref[l], be1_ref[l])
        ff = jnp.maximum(_mm(xb.reshape(BB * N, IN), f1t_ref[l]) + fb1_ref[l], 0.0)
        y = _mm(ff, f2t_ref[l]) + fb2_ref[l]
        xb = _ln(xb + y.reshape(BB, N, IN), g2_ref[l], be2_ref[l])
    o_ref[...] = jnp.maximum(xb, 0.0)


def _proj_body(xf_ref, w_ref, b_ref, o_ref):
    o_ref[...] = _mm(xf_ref[...], w_ref[...]) + b_ref[...]


@functools.partial(jax.jit)
def kernel(x, adj_matrix, Wl, bl, al, ff_w1, ff_b1, ff_w2, ff_b2,
           ln1_g, ln1_b, ln2_g, ln2_b, w_out, b_out):
    wt = Wl.transpose(0, 1, 3, 2)
    blr = bl.reshape(L, H, 1, E)
    a1t = al[:, :, :E, :].transpose(0, 1, 3, 2)
    a2 = al[:, :, E:, :]
    f1t = ff_w1.transpose(0, 2, 1)
    fb1 = ff_b1.reshape(L, 1, FF)
    f2t = ff_w2.transpose(0, 2, 1)
    fb2 = ff_b2.reshape(L, 1, IN)
    g1 = ln1_g.reshape(L, 1, 1, IN)
    be1 = ln1_b.reshape(L, 1, 1, IN)
    g2 = ln2_g.reshape(L, 1, 1, IN)
    be2 = ln2_b.reshape(L, 1, 1, IN)

    full = lambda shape: pl.BlockSpec(shape, lambda i: (0,) * len(shape))
    xr = pl.pallas_call(
        _ghat_body,
        grid=(B // BB,),
        in_specs=[
            pl.BlockSpec((BB, N, IN), lambda i: (i, 0, 0)),
            full((N, N)),
            full((L, H, IN, E)),
            full((L, H, 1, E)),
            full((L, H, N, E)),
            full((L, H, E, N)),
            full((L, IN, FF)),
            full((L, 1, FF)),
            full((L, FF, IN)),
            full((L, 1, IN)),
            full((L, 1, 1, IN)),
            full((L, 1, 1, IN)),
            full((L, 1, 1, IN)),
            full((L, 1, 1, IN)),
        ],
        out_specs=pl.BlockSpec((BB, N, IN), lambda i: (i, 0, 0)),
        out_shape=jax.ShapeDtypeStruct((B, N, IN), jnp.float32),
        compiler_params=pltpu.CompilerParams(
            dimension_semantics=("parallel",)),
    )(x, adj_matrix, wt, blr, a1t, a2, f1t, fb1, f2t, fb2, g1, be1, g2, be2)

    xf = xr.reshape(B, N * IN)
    out = pl.pallas_call(
        _proj_body,
        grid=(B // BBO,),
        in_specs=[
            pl.BlockSpec((BBO, N * IN), lambda i: (i, 0)),
            full((N * IN, OUT)),
            full((1, OUT)),
        ],
        out_specs=pl.BlockSpec((BBO, OUT), lambda i: (i, 0)),
        out_shape=jax.ShapeDtypeStruct((B, OUT), jnp.float32),
        compiler_params=pltpu.CompilerParams(
            dimension_semantics=("parallel",)),
    )(xf, w_out.T, b_out.reshape(1, OUT))
    return out
